# Initial kernel scaffold; baseline (speedup 1.0000x reference)
import jax
import jax.numpy as jnp
from jax.experimental import pallas as pl


def kernel(x, edge_index, edge_weight, W1, b1, W2, b2):
    raise NotImplementedError("write your pallas kernel here")


# speedup vs baseline: 11.7611x; 11.7611x over previous
import functools
import math

import jax
import jax.numpy as jnp
from jax import lax
from jax.experimental import pallas as pl
from jax.experimental.pallas import tpu as pltpu
from jax.experimental.pallas import tpu_sc as plsc

N, E, D, H = 10000, 160000, 256, 256
HH = H // 2
NTILES = 32
EPC = 128
EROWS = math.ceil(E / (NTILES * EPC)) * NTILES
EPAD = EROWS * EPC
CPT = EROWS // 16
NPAD = 10240
NSL = NPAD // NTILES
RPT = NPAD // 16

_mesh = plsc.VectorSubcoreMesh(core_axis_name="c", subcore_axis_name="s")
_sc_params = pltpu.CompilerParams(needs_layout_passes=False)


@functools.partial(
    pl.kernel, mesh=_mesh,
    out_type=[jax.ShapeDtypeStruct((NPAD,), jnp.float32),
              jax.ShapeDtypeStruct((2 * EROWS, EPC), jnp.float32)],
    compiler_params=_sc_params,
    scratch_types=[
        pltpu.VMEM((CPT, EPC), jnp.int32),
        pltpu.VMEM((CPT, EPC), jnp.int32),
        pltpu.VMEM((CPT, EPC), jnp.float32),
        pltpu.VMEM((CPT, EPC), jnp.float32),
        pltpu.VMEM((NPAD // 16,), jnp.float32),
        pltpu.VMEM((NPAD,), jnp.float32),
        pltpu.VMEM((NPAD,), jnp.float32),
        pltpu.VMEM_SHARED((NPAD,), jnp.float32),
    ],
)
def _deg_dinv(src_hbm, dst_hbm, ew_hbm, dinv_hbm, w_hbm,
              src_v, dst_v, ew_v, w_v, ones_v, deg_v, dinv_v, deg_sh):
    c = lax.axis_index("c")
    s = lax.axis_index("s")
    nsh = NPAD // 16

    def _ones(i, _):
        ones_v[pl.ds(i * 16, 16)] = jnp.ones((16,), jnp.float32)
        return 0
    lax.fori_loop(0, nsh // 16, _ones, 0, unroll=4)
    pltpu.sync_copy(ones_v, deg_sh.at[pl.ds(s * nsh, nsh)])
    plsc.subcore_barrier()

    pltpu.sync_copy(src_hbm.at[pl.ds(s * CPT, CPT)], src_v)
    pltpu.sync_copy(dst_hbm.at[pl.ds(s * CPT, CPT)], dst_v)
    pltpu.sync_copy(ew_hbm.at[pl.ds(s * CPT, CPT)], ew_v)

    def _scat(j, _):
        pltpu.sync_copy(ew_v.at[j], deg_sh.at[dst_v.at[j]], add=True)
        return 0
    lax.fori_loop(0, CPT, _scat, 0)
    plsc.subcore_barrier()

    pltpu.sync_copy(deg_sh, deg_v)

    def _rsqrt(i, _):
        sl = pl.ds(i * 16, 16)
        x = deg_v[sl]
        xi = lax.bitcast_convert_type(x, jnp.int32)
        y = lax.bitcast_convert_type(jnp.int32(0x5F3759DF) - (xi >> 1),
                                     jnp.float32)
        y = y * (1.5 - 0.5 * x * y * y)
        y = y * (1.5 - 0.5 * x * y * y)
        y = y * (1.5 - 0.5 * x * y * y)
        dinv_v[sl] = y
        return 0
    lax.fori_loop(0, NPAD // 16, _rsqrt, 0, unroll=2)

    off = (c * 16 + s) * NSL
    pltpu.sync_copy(dinv_v.at[pl.ds(off, NSL)], dinv_hbm.at[pl.ds(off, NSL)])

    def _wrow(j, _):
        for g in range(8):
            sl = pl.ds(g * 16, 16)
            w_v[j, sl] = ew_v[j, sl] * plsc.load_gather(dinv_v,
                                                        [src_v[j, sl]])
        return 0
    lax.fori_loop(0, CPT, _wrow, 0)
    pltpu.sync_copy(w_v, w_hbm.at[pl.ds(c * EROWS + s * CPT, CPT)])


@functools.partial(
    pl.kernel, mesh=_mesh,
    out_type=jax.ShapeDtypeStruct((2 * NPAD, HH), jnp.float32),
    compiler_params=_sc_params,
    scratch_types=[
        pltpu.VMEM((CPT, EPC), jnp.int32),
        pltpu.VMEM((CPT, EPC), jnp.int32),
        pltpu.VMEM((CPT, EPC), jnp.float32),
        pltpu.VMEM((EPC,), jnp.int32),
        pltpu.VMEM((EPC, HH), jnp.float32),
        pltpu.VMEM_SHARED((NPAD, HH), jnp.float32),
        pltpu.SemaphoreType.DMA,
    ],
)
def _message(src_hbm, dst_hbm, w_hbm, h_hbm, agg_hbm,
             src_v, dst_v, w_v, idx_v, rows_v, agg_sh, sem):
    c = lax.axis_index("c")
    s = lax.axis_index("s")

    def _z(r, _):
        for g in range(8):
            rows_v[r, pl.ds(g * 16, 16)] = jnp.zeros((16,), jnp.float32)
        return 0
    lax.fori_loop(0, EPC, _z, 0)
    for k in range(RPT // EPC):
        pltpu.sync_copy(rows_v, agg_sh.at[pl.ds(s * RPT + k * EPC, EPC)])
    plsc.subcore_barrier()

    pltpu.sync_copy(src_hbm.at[pl.ds(s * CPT, CPT)], src_v)
    pltpu.sync_copy(dst_hbm.at[pl.ds(s * CPT, CPT)], dst_v)
    pltpu.sync_copy(w_hbm.at[pl.ds(s * CPT, CPT)], w_v)
    cN = c * N

    def _chunk(j, _):
        for g in range(8):
            sl = pl.ds(g * 16, 16)
            idx_v[sl] = src_v[j, sl] + cN

        pltpu.async_copy(h_hbm.at[idx_v], rows_v, sem).wait()

        def _erow(e, _):
            we = plsc.load_gather(
                w_v, [lax.broadcast(j, (16,)), lax.broadcast(e, (16,))])
            for g in range(8):
                slg = pl.ds(g * 16, 16)
                rows_v[e, slg] = rows_v[e, slg] * we
            return 0
        lax.fori_loop(0, EPC, _erow, 0, unroll=2)

        pltpu.sync_copy(rows_v, agg_sh.at[dst_v.at[j]], add=True)
        return 0
    lax.fori_loop(0, CPT, _chunk, 0)
    plsc.subcore_barrier()

    for k in range(RPT // EPC):
        off = s * RPT + k * EPC
        pltpu.sync_copy(agg_sh.at[pl.ds(off, EPC)], rows_v)
        pltpu.sync_copy(rows_v, agg_hbm.at[pl.ds(c * NPAD + off, EPC)])


def _mm_body(x_ref, w1_ref, o_ref):
    o_ref[...] = jnp.dot(x_ref[...], w1_ref[...],
                         preferred_element_type=jnp.float32)


def _head_body(a0_ref, a1_ref, h0_ref, h1_ref, di_ref, b1_ref, w2_ref, b2_ref,
               o_ref):
    di = di_ref[...]
    z0 = jnp.maximum(di * (a0_ref[...] + di * h0_ref[...])
                     + b1_ref[0:1, :HH], 0.0)
    z1 = jnp.maximum(di * (a1_ref[...] + di * h1_ref[...])
                     + b1_ref[0:1, HH:], 0.0)
    acc = jnp.dot(z0, w2_ref[:HH, :], preferred_element_type=jnp.float32)
    acc += jnp.dot(z1, w2_ref[HH:, :], preferred_element_type=jnp.float32)
    o_ref[...] = acc + b2_ref[0:1, :]


def kernel(x, edge_index, edge_weight, W1, b1, W2, b2):
    src = edge_index[0]
    dst = edge_index[1]
    pad = EPAD - E
    fill = jnp.arange(pad, dtype=src.dtype)
    src_p = jnp.concatenate([src, fill]).reshape(EROWS, EPC)
    dst_p = jnp.concatenate([dst, fill]).reshape(EROWS, EPC)
    ew_p = jnp.concatenate(
        [edge_weight, jnp.zeros((pad,), edge_weight.dtype)]).reshape(EROWS, EPC)

    dinv_pad, w_full = _deg_dinv(src_p, dst_p, ew_p)
    w2d = w_full[:EROWS]

    blk = 1000
    nblk = N // blk
    h_stack = pl.pallas_call(
        _mm_body,
        grid=(nblk, 2),
        in_specs=[
            pl.BlockSpec((blk, D), lambda i, j: (i, 0)),
            pl.BlockSpec((D, HH), lambda i, j: (0, j)),
        ],
        out_specs=pl.BlockSpec((blk, HH), lambda i, j: (j * nblk + i, 0)),
        out_shape=jax.ShapeDtypeStruct((2 * N, HH), jnp.float32),
    )(x, W1)

    agg_full = _message(src_p, dst_p, w2d, h_stack)
    agg0 = agg_full[:N]
    agg1 = agg_full[NPAD:NPAD + N]

    dinv = dinv_pad[:N].reshape(N, 1)
    out = pl.pallas_call(
        _head_body,
        grid=(nblk,),
        in_specs=[
            pl.BlockSpec((blk, HH), lambda i: (i, 0)),
            pl.BlockSpec((blk, HH), lambda i: (i, 0)),
            pl.BlockSpec((blk, HH), lambda i: (i, 0)),
            pl.BlockSpec((blk, HH), lambda i: (i + nblk, 0)),
            pl.BlockSpec((blk, 1), lambda i: (i, 0)),
            pl.BlockSpec((1, H), lambda i: (0, 0)),
            pl.BlockSpec((H, 1), lambda i: (0, 0)),
            pl.BlockSpec((1, 1), lambda i: (0, 0)),
        ],
        out_specs=pl.BlockSpec((blk, 1), lambda i: (i, 0)),
        out_shape=jax.ShapeDtypeStruct((N, 1), jnp.float32),
    )(agg0, agg1, h_stack, h_stack, dinv, b1.reshape(1, H), W2,
      b2.reshape(1, 1))
    return out[:, 0]
